# Initial kernel scaffold; baseline (speedup 1.0000x reference)
import jax
import jax.numpy as jnp
from jax.experimental import pallas as pl


def kernel(node_type, velocity, mesh_pos, world_pos, srcs, dsts, wsrcs, wdsts, params):
    raise NotImplementedError("write your pallas kernel here")


# speedup vs baseline: 2.2154x; 2.2154x over previous
import functools

import jax
import jax.numpy as jnp
from jax import lax
from jax.experimental import pallas as pl
from jax.experimental.pallas import tpu as pltpu
from jax.experimental.pallas import tpu_sc as plsc

LAT = 128
_NC, _NS = 2, 16
_NW = _NC * _NS
_CW = 128
_EB = 2000
_NB = 1000
_F32 = jnp.float32


def _sc_gather(table, idx):
  n, w = table.shape
  e = idx.shape[0]
  assert e % _CW == 0
  nch = e // _CW
  mesh = plsc.VectorSubcoreMesh(core_axis_name="c", subcore_axis_name="s")

  @functools.partial(
      pl.kernel,
      out_type=jax.ShapeDtypeStruct((e, w), table.dtype),
      mesh=mesh,
      compiler_params=pltpu.CompilerParams(use_tc_tiling_on_sc=(w % 128 == 0)),
      scratch_types=[
          pltpu.VMEM((1, _CW), jnp.int32),
          pltpu.VMEM((_CW, w), table.dtype),
      ],
  )
  def k(tab_hbm, idx_hbm, out_hbm, ibuf, rbuf):
    wid = lax.axis_index("s") * _NC + lax.axis_index("c")

    @pl.loop(wid, nch, step=_NW)
    def _(c):
      pltpu.sync_copy(idx_hbm.at[0, pl.ds(c * _CW, _CW)], ibuf.at[0])
      pltpu.sync_copy(tab_hbm.at[ibuf.at[0]], rbuf)
      pltpu.sync_copy(rbuf, out_hbm.at[pl.ds(c * _CW, _CW)])

  return k(table, idx.reshape(1, e))


def _sc_scatter_add(x, idx, n_out):
  e, w = x.shape
  assert e % _CW == 0
  nch = e // _CW
  zr = 80
  assert n_out % zr == 0
  nzc = n_out // zr
  mesh = plsc.VectorSubcoreMesh(core_axis_name="c", subcore_axis_name="s")

  @functools.partial(
      pl.kernel,
      out_type=jax.ShapeDtypeStruct((_NC, n_out, w), _F32),
      mesh=mesh,
      scratch_types=[
          pltpu.VMEM_SHARED((n_out, w), _F32),
          pltpu.VMEM((zr, w), _F32),
          pltpu.VMEM((1, _CW), jnp.int32),
          pltpu.VMEM((_CW, w), _F32),
      ],
  )
  def k(x_hbm, idx_hbm, out_hbm, acc, zbuf, ibuf, xbuf):
    cid = lax.axis_index("c")
    sid = lax.axis_index("s")
    wid = sid * _NC + cid

    @pl.loop(0, zr)
    def _(r):
      for j in range(w // 16):
        zbuf[r, pl.ds(j * 16, 16)] = jnp.zeros((16,), _F32)

    @pl.loop(sid, nzc, step=_NS)
    def _(c):
      pltpu.sync_copy(zbuf, acc.at[pl.ds(c * zr, zr)])

    plsc.subcore_barrier()

    @pl.loop(wid, nch, step=_NW)
    def _(c):
      pltpu.sync_copy(idx_hbm.at[0, pl.ds(c * _CW, _CW)], ibuf.at[0])
      pltpu.sync_copy(x_hbm.at[pl.ds(c * _CW, _CW)], xbuf)
      pltpu.sync_copy(xbuf, acc.at[ibuf.at[0]], add=True)

    plsc.subcore_barrier()

    @pl.loop(sid, nzc, step=_NS)
    def _(c):
      pltpu.sync_copy(acc.at[pl.ds(c * zr, zr)],
                      out_hbm.at[cid, pl.ds(c * zr, zr)])

  return k(x, idx.reshape(1, e))


def _mm(x, w):
  return lax.dot_general(x, w, (((1,), (0,)), ((), ())),
                         preferred_element_type=_F32,
                         precision=lax.Precision.HIGHEST)


def _ln(h, cst):
  mu = jnp.mean(h, axis=1, keepdims=True)
  d = h - mu
  var = jnp.mean(d * d, axis=1, keepdims=True)
  return d / jnp.sqrt(var + 1e-5) * cst[3:4] + cst[4:5]


def _mlp_tail(h0, w1, w2, cst):
  h = jnp.maximum(_mm(h0, w1) + cst[1:2], 0.0)
  return _mm(h, w2) + cst[2:3]


def _edge_body(em_ref, sa_ref, sb_ref, w_ref, c_ref, emn_ref, emo_ref):
  x = em_ref[...]
  cst = c_ref[...]
  h0 = jnp.maximum(_mm(x, w_ref[0:128]) + sa_ref[...] + sb_ref[...] + cst[0:1],
                   0.0)
  y = _ln(_mlp_tail(h0, w_ref[128:256], w_ref[256:384], cst), cst)
  emn_ref[...] = y
  emo_ref[...] = x + y


def _node_body(v_ref, am_ref, aw_ref, w_ref, c_ref, vo_ref):
  x = v_ref[...]
  cst = c_ref[...]
  am = am_ref[0] + am_ref[1]
  aw = aw_ref[0] + aw_ref[1]
  h0 = jnp.maximum(
      _mm(x, w_ref[0:128]) + _mm(am, w_ref[128:256]) + _mm(aw, w_ref[256:384])
      + cst[0:1], 0.0)
  y = _ln(_mlp_tail(h0, w_ref[384:512], w_ref[512:640], cst), cst)
  vo_ref[...] = x + y


def _proj_body(v_ref, w_ref, o0_ref, o1_ref, o2_ref, o3_ref):
  r = _mm(v_ref[...], w_ref[...])
  o0_ref[...] = r[:, 0:128]
  o1_ref[...] = r[:, 128:256]
  o2_ref[...] = r[:, 256:384]
  o3_ref[...] = r[:, 384:512]


def _menc_body(ta_ref, tb_ref, w_ref, c_ref, o_ref):
  d = ta_ref[...] - tb_ref[...]
  s = d * d
  n1 = jnp.sqrt(s[:, 0:1] + s[:, 1:2] + s[:, 2:3])
  n2 = jnp.sqrt(s[:, 3:4] + s[:, 4:5] + s[:, 5:6])
  x = jnp.concatenate(
      [d[:, 0:6], n1, n2, jnp.zeros((d.shape[0], 8), _F32)], axis=1)
  cst = c_ref[...]
  h0 = jnp.maximum(_mm(x, w_ref[0:16]) + cst[0:1], 0.0)
  o_ref[...] = _ln(_mlp_tail(h0, w_ref[16:144], w_ref[144:272], cst), cst)


def _wenc_body(ta_ref, tb_ref, w_ref, c_ref, o_ref):
  d = ta_ref[...] - tb_ref[...]
  s = d * d
  n = jnp.sqrt(s[:, 0:1] + s[:, 1:2] + s[:, 2:3])
  x = jnp.concatenate(
      [d[:, 0:3], n, jnp.zeros((d.shape[0], 12), _F32)], axis=1)
  cst = c_ref[...]
  h0 = jnp.maximum(_mm(x, w_ref[0:16]) + cst[0:1], 0.0)
  o_ref[...] = _ln(_mlp_tail(h0, w_ref[16:144], w_ref[144:272], cst), cst)


def _nenc_body(vel_ref, nt_ref, w_ref, c_ref, o_ref):
  nt = nt_ref[...][:, 0:9]
  oh = (nt == lax.broadcasted_iota(jnp.int32, nt.shape, 1)).astype(_F32)
  x = jnp.concatenate(
      [vel_ref[...][:, 0:3], oh, jnp.zeros((oh.shape[0], 4), _F32)], axis=1)
  cst = c_ref[...]
  h0 = jnp.maximum(_mm(x, w_ref[0:16]) + cst[0:1], 0.0)
  o_ref[...] = _ln(_mlp_tail(h0, w_ref[16:144], w_ref[144:272], cst), cst)


def _dec_body(v_ref, w_ref, c_ref, o_ref):
  cst = c_ref[...]
  h = jnp.maximum(_mm(v_ref[...], w_ref[0:128]) + cst[0:1], 0.0)
  h = jnp.maximum(_mm(h, w_ref[128:256]) + cst[1:2], 0.0)
  o_ref[...] = _mm(h, w_ref[256:384]) + cst[2:3]


def _row_spec(rb, cols=LAT):
  return pl.BlockSpec((rb, cols), lambda i: (i, 0))


def _const_spec(shape):
  return pl.BlockSpec(shape, lambda i: (0,) * len(shape))


def _edge_mlp(em, sa, sb, wall, cst):
  e = em.shape[0]
  return pl.pallas_call(
      _edge_body,
      grid=(e // _EB,),
      in_specs=[_row_spec(_EB)] * 3 + [_const_spec(wall.shape),
                                       _const_spec(cst.shape)],
      out_specs=[_row_spec(_EB)] * 2,
      out_shape=[jax.ShapeDtypeStruct((e, LAT), _F32)] * 2,
  )(em, sa, sb, wall, cst)


def _node_mlp(v, aggm, aggw, wall, cst):
  n = v.shape[0]
  agg_spec = pl.BlockSpec((2, _NB, LAT), lambda i: (0, i, 0))
  return pl.pallas_call(
      _node_body,
      grid=(n // _NB,),
      in_specs=[_row_spec(_NB), agg_spec, agg_spec,
                _const_spec(wall.shape), _const_spec(cst.shape)],
      out_specs=_row_spec(_NB),
      out_shape=jax.ShapeDtypeStruct((n, LAT), _F32),
  )(v, aggm, aggw, wall, cst)


def _node_proj(v, wcat):
  n = v.shape[0]
  return pl.pallas_call(
      _proj_body,
      grid=(n // _NB,),
      in_specs=[_row_spec(_NB), _const_spec(wcat.shape)],
      out_specs=[_row_spec(_NB)] * 4,
      out_shape=[jax.ShapeDtypeStruct((n, LAT), _F32)] * 4,
  )(v, wcat)


def _encoder(body, ta, tb, wall, cst, rb):
  e = ta.shape[0]
  return pl.pallas_call(
      body,
      grid=(e // rb,),
      in_specs=[_row_spec(rb, 16), _row_spec(rb, 16),
                _const_spec(wall.shape), _const_spec(cst.shape)],
      out_specs=_row_spec(rb),
      out_shape=jax.ShapeDtypeStruct((e, LAT), _F32),
  )(ta, tb, wall, cst)


def _node_encoder(vel16, nt16, wall, cst):
  n = vel16.shape[0]
  return pl.pallas_call(
      _nenc_body,
      grid=(n // _NB,),
      in_specs=[_row_spec(_NB, 16), _row_spec(_NB, 16),
                _const_spec(wall.shape), _const_spec(cst.shape)],
      out_specs=_row_spec(_NB),
      out_shape=jax.ShapeDtypeStruct((n, LAT), _F32),
  )(vel16, nt16, wall, cst)


def _decoder(v, wall, cst):
  n = v.shape[0]
  return pl.pallas_call(
      _dec_body,
      grid=(n // _NB,),
      in_specs=[_row_spec(_NB), _const_spec(wall.shape),
                _const_spec(cst.shape)],
      out_specs=_row_spec(_NB),
      out_shape=jax.ShapeDtypeStruct((n, LAT), _F32),
  )(v, wall, cst)


def _cst(p, pad_out=False):
  b2 = p['b2']
  if pad_out:
    b2 = jnp.pad(b2, (0, LAT - b2.shape[0]))
  g = p.get('g', jnp.zeros((LAT,), _F32))
  bln = p.get('bln', jnp.zeros((LAT,), _F32))
  return jnp.stack([p['b0'], p['b1'], b2, g, bln])


def _enc_wall(p, row_order):
  w0 = p['W0'][jnp.array(row_order)]
  w0 = jnp.pad(w0, ((0, 16 - w0.shape[0]), (0, 0)))
  return jnp.concatenate([w0, p['W1'], p['W2']], axis=0)


def kernel(node_type, velocity, mesh_pos, world_pos, srcs, dsts, wsrcs, wdsts,
           params):
  n = node_type.shape[0]
  vel = velocity[0]
  mp = mesh_pos[0]
  wp = world_pos[0]
  srcs = srcs.astype(jnp.int32)
  dsts = dsts.astype(jnp.int32)
  wsrcs = wsrcs.astype(jnp.int32)
  wdsts = wdsts.astype(jnp.int32)

  z10 = jnp.zeros((n, 10), _F32)
  ts = jnp.concatenate([mp, wp, z10], axis=1)
  td = jnp.concatenate([mp, mp, z10], axis=1)
  tw = jnp.concatenate([wp, jnp.zeros((n, 13), _F32)], axis=1)

  gsm = _sc_gather(ts, srcs)
  gdm = _sc_gather(td, dsts)
  gsw = _sc_gather(tw, wsrcs)
  gdw = _sc_gather(tw, wdsts)

  vel16 = jnp.concatenate([vel, jnp.zeros((n, 13), _F32)], axis=1)
  nt16 = jnp.broadcast_to(node_type.astype(jnp.int32)[:, None], (n, 16))
  v = _node_encoder(vel16, nt16, _enc_wall(params['node_enc'], range(12)),
                    _cst(params['node_enc']))
  em = _encoder(_menc_body, gsm, gdm,
                _enc_wall(params['mesh_enc'], [0, 1, 2, 4, 5, 6, 3, 7]),
                _cst(params['mesh_enc']), _EB)
  ew = _encoder(_wenc_body, gsw, gdw,
                _enc_wall(params['world_enc'], range(4)),
                _cst(params['world_enc']), _EB)

  for sp in params['mp']:
    w0m, w0w, w0n = sp['me']['W0'], sp['we']['W0'], sp['nd']['W0']
    wproj = jnp.concatenate(
        [w0m[128:256], w0m[256:384], w0w[128:256], w0w[256:384]], axis=1)
    pm, qm, pw, qw = _node_proj(v, wproj)
    sam = _sc_gather(pm, srcs)
    sbm = _sc_gather(qm, dsts)
    saw = _sc_gather(pw, wsrcs)
    sbw = _sc_gather(qw, wdsts)
    wall_m = jnp.concatenate([w0m[0:128], sp['me']['W1'], sp['me']['W2']], 0)
    wall_w = jnp.concatenate([w0w[0:128], sp['we']['W1'], sp['we']['W2']], 0)
    em_n, em = _edge_mlp(em, sam, sbm, wall_m, _cst(sp['me']))
    ew_n, ew = _edge_mlp(ew, saw, sbw, wall_w, _cst(sp['we']))
    aggm = _sc_scatter_add(em_n, dsts, n)
    aggw = _sc_scatter_add(ew_n, wdsts, n)
    wall_n = jnp.concatenate([w0n, sp['nd']['W1'], sp['nd']['W2']], axis=0)
    v = _node_mlp(v, aggm, aggw, wall_n, _cst(sp['nd']))

  wdec = jnp.concatenate(
      [params['dec']['W0'], params['dec']['W1'],
       jnp.pad(params['dec']['W2'], ((0, 0), (0, LAT - 3)))], axis=0)
  out = _decoder(v, wdec, _cst(params['dec'], pad_out=True))
  return out[None, :, 0:3]
